# baseline (speedup 1.0000x reference)
import functools

import jax
import jax.numpy as jnp
from jax import lax
from jax.experimental import pallas as pl
from jax.experimental.pallas import tpu as pltpu
from jax.experimental.pallas import tpu_sc as plsc

NC = 2
NS = 16
NW = NC * NS

D = 64
B = 16384 * 50
B_PER_W = B // NW
CHUNK = 400
N_CHUNKS = B_PER_W // CHUNK
NBUF = 4
MAIN_G = (N_CHUNKS - NBUF) // NBUF
assert B_PER_W % CHUNK == 0 and N_CHUNKS % NBUF == 0 and CHUNK % 8 == 0

_mesh = plsc.VectorSubcoreMesh(core_axis_name="c", subcore_axis_name="s")


@functools.partial(
    pl.kernel,
    out_type=jax.ShapeDtypeStruct((B, D), jnp.float32),
    mesh=_mesh,
    scratch_types=(
        [pltpu.VMEM((B_PER_W,), jnp.int32)]
        + [pltpu.VMEM((CHUNK, D), jnp.float32) for _ in range(NBUF)]
        + [pltpu.SemaphoreType.DMA for _ in range(2 * NBUF)]
    ),
    compiler_params=pltpu.CompilerParams(use_tc_tiling_on_sc=False),
)
def _gather_kernel(idx_hbm, table_hbm, out_hbm, idx_v, *scratch):
    row_bufs = scratch[:NBUF]
    gsems = scratch[NBUF:2 * NBUF]
    osems = scratch[2 * NBUF:]

    wid = lax.axis_index("s") * NC + lax.axis_index("c")
    wbase = wid * B_PER_W

    def out_slice(i):
        return pl.ds(pl.multiple_of(wbase + i * CHUNK, 8), CHUNK)

    def idx_slice(i):
        return pl.ds(pl.multiple_of(i * CHUNK, 8), CHUNK)

    pltpu.sync_copy(idx_hbm.at[pl.ds(pl.multiple_of(wbase, 8), B_PER_W)], idx_v)

    for b in range(NBUF):
        pltpu.async_copy(
            table_hbm.at[idx_v.at[idx_slice(b)]], row_bufs[b], gsems[b])

    @pl.loop(0, MAIN_G)
    def main(g):
        for b in range(NBUF):
            i = g * NBUF + b
            pltpu.make_async_copy(
                table_hbm.at[idx_v.at[idx_slice(i)]], row_bufs[b],
                gsems[b]).wait()
            pltpu.async_copy(row_bufs[b], out_hbm.at[out_slice(i)], osems[b])
            pltpu.make_async_copy(
                row_bufs[b], out_hbm.at[out_slice(i)], osems[b]).wait()
            pltpu.async_copy(
                table_hbm.at[idx_v.at[idx_slice(i + NBUF)]], row_bufs[b],
                gsems[b])

    for b in range(NBUF):
        i = MAIN_G * NBUF + b
        pltpu.make_async_copy(
            table_hbm.at[idx_v.at[idx_slice(i)]], row_bufs[b], gsems[b]).wait()
        pltpu.async_copy(row_bufs[b], out_hbm.at[out_slice(i)], osems[b])
    for b in range(NBUF):
        i = MAIN_G * NBUF + b
        pltpu.make_async_copy(
            row_bufs[b], out_hbm.at[out_slice(i)], osems[b]).wait()


def kernel(token_ids, weight):
    flat = token_ids.reshape(-1).astype(jnp.int32)
    out = _gather_kernel(flat, weight)
    return out.reshape(token_ids.shape + (weight.shape[1],))
